# baseline (speedup 1.0000x reference)
import jax
import jax.numpy as jnp
from jax import lax
from jax.experimental import pallas as pl
from jax.experimental.pallas import tpu as pltpu
from jax.experimental.pallas import tpu_sc as plsc

N = 10000
E = 320000
D = 128
NC = 2
NS = 16
NW = NC * NS
CHUNK = 128
NCHT = 2560
EP = NCHT * CHUNK
CW = NCHT // NW
NPAD = N + 16
RPS = 624
RPS_LAST = N - 15 * RPS

_mesh = plsc.VectorSubcoreMesh(
    core_axis_name="c", subcore_axis_name="s", num_cores=NC, num_subcores=NS
)


def _sc_segsum_wide(x_hbm, src_hbm, dst_hbm, out_hbm, acc,
                    sidx0, sidx1, sidx2, didx0, didx1, didx2,
                    rows0, rows1, rows2,
                    isem0, isem1, isem2, jsem0, jsem1, jsem2,
                    gsem0, gsem1, gsem2, ssem0, ssem1, ssem2):
    c = lax.axis_index("c")
    s = lax.axis_index("s")
    wid = s * NC + c
    e0 = wid * CW * CHUNK

    sidx = (sidx0, sidx1, sidx2)
    didx = (didx0, didx1, didx2)
    rows = (rows0, rows1, rows2)
    isem = (isem0, isem1, isem2)
    jsem = (jsem0, jsem1, jsem2)
    gsem = (gsem0, gsem1, gsem2)
    ssem = (ssem0, ssem1, ssem2)

    zero16 = jnp.zeros((16,), jnp.float32)

    def _zrow(i, carry):
        for k in range(D // 16):
            rows0[i, pl.ds(k * 16, 16)] = zero16
        return carry

    lax.fori_loop(0, CHUNK, _zrow, 0)
    r0 = s * RPS

    @pl.when(s < NS - 1)
    def _():
        off = 0
        for m in (128, 128, 128, 128, RPS - 4 * 128):
            pltpu.sync_copy(rows0.at[pl.ds(0, m)],
                            acc.at[pl.ds(r0 + off, m)])
            off += m

    @pl.when(s == NS - 1)
    def _():
        for k in range(5):
            pltpu.sync_copy(rows0, acc.at[pl.ds(15 * RPS + k * CHUNK, CHUNK)])
        pltpu.sync_copy(rows0.at[pl.ds(0, NPAD - N)],
                        acc.at[pl.ds(N, NPAD - N)])

    plsc.subcore_barrier()

    def idx_issue(g, b):
        pltpu.async_copy(src_hbm.at[pl.ds(e0 + g * CHUNK, CHUNK)],
                         sidx[b], isem[b])
        pltpu.async_copy(dst_hbm.at[pl.ds(e0 + g * CHUNK, CHUNK)],
                         didx[b], jsem[b])

    def idx_wait(b):
        pltpu.make_async_copy(src_hbm.at[pl.ds(e0, CHUNK)], sidx[b],
                              isem[b]).wait()
        pltpu.make_async_copy(dst_hbm.at[pl.ds(e0, CHUNK)], didx[b],
                              jsem[b]).wait()

    def gather_issue(b):
        pltpu.async_copy(x_hbm.at[sidx[b]], rows[b], gsem[b])

    def gather_wait(b):
        pltpu.make_async_copy(x_hbm.at[sidx[b]], rows[b], gsem[b]).wait()

    def scatter_issue(b):
        pltpu.async_copy(rows[b], acc.at[didx[b]], ssem[b], add=True)

    def scatter_wait(b):
        pltpu.make_async_copy(rows[b], acc.at[didx[b]], ssem[b]).wait()

    idx_issue(0, 0)
    idx_issue(1, 1)
    idx_wait(0)
    gather_issue(0)
    gather_wait(0)
    scatter_issue(0)
    idx_issue(2, 2)
    idx_wait(1)
    gather_issue(1)

    def _steady(outer, carry):
        g1 = 1 + outer * 3
        for u in range(3):
            g = g1 + u
            b0 = (1 + u) % 3
            b2 = (u + 3) % 3
            b1 = (2 + u) % 3
            gather_wait(b0)
            scatter_issue(b0)
            scatter_wait(b2)
            idx_issue(g + 2, b2)
            idx_wait(b1)
            gather_issue(b1)
        return carry

    lax.fori_loop(0, 25, _steady, 0)

    for g in range(76, CW):
        b0 = g % 3
        b2 = (g + 2) % 3
        b1 = (g + 1) % 3
        gather_wait(b0)
        scatter_issue(b0)
        scatter_wait(b2)
        if g + 2 < CW:
            idx_issue(g + 2, b2)
        if g + 1 < CW:
            idx_wait(b1)
            gather_issue(b1)
    scatter_wait((CW - 1) % 3)

    plsc.subcore_barrier()

    @pl.when(s < NS - 1)
    def _():
        pltpu.sync_copy(acc.at[pl.ds(r0, RPS)], out_hbm.at[c, pl.ds(r0, RPS)])

    @pl.when(s == NS - 1)
    def _():
        pltpu.sync_copy(acc.at[pl.ds(15 * RPS, RPS_LAST)],
                        out_hbm.at[c, pl.ds(15 * RPS, RPS_LAST)])


_sc1 = pl.kernel(
    _sc_segsum_wide,
    out_type=jax.ShapeDtypeStruct((NC, N, D), jnp.float32),
    mesh=_mesh,
    scratch_types=[
        pltpu.VMEM_SHARED((NPAD, D), jnp.float32),
        pltpu.VMEM((CHUNK,), jnp.int32),
        pltpu.VMEM((CHUNK,), jnp.int32),
        pltpu.VMEM((CHUNK,), jnp.int32),
        pltpu.VMEM((CHUNK,), jnp.int32),
        pltpu.VMEM((CHUNK,), jnp.int32),
        pltpu.VMEM((CHUNK,), jnp.int32),
        pltpu.VMEM((CHUNK, D), jnp.float32),
        pltpu.VMEM((CHUNK, D), jnp.float32),
        pltpu.VMEM((CHUNK, D), jnp.float32),
    ] + [pltpu.SemaphoreType.DMA] * 12,
)


def _sc_segsum_scalar(y_hbm, src_hbm, dst_hbm, out_hbm, acc, sidx, didx,
                      yv0, yv1, yv2, yv3, zbuf,
                      gsem0, gsem1, gsem2, gsem3,
                      ssem0, ssem1, ssem2, ssem3):
    c = lax.axis_index("c")
    s = lax.axis_index("s")
    wid = s * NC + c

    zero16 = jnp.zeros((16,), jnp.float32)

    def _z(i, carry):
        zbuf[pl.ds(i * 16, 16)] = zero16
        return carry

    lax.fori_loop(0, RPS_LAST // 16, _z, 0)

    @pl.when(s < NS - 1)
    def _():
        pltpu.sync_copy(zbuf.at[pl.ds(0, RPS)], acc.at[pl.ds(s * RPS, RPS)])

    @pl.when(s == NS - 1)
    def _():
        pltpu.sync_copy(zbuf, acc.at[pl.ds(15 * RPS, RPS_LAST)])
        pltpu.sync_copy(zbuf.at[pl.ds(0, NPAD - N)], acc.at[pl.ds(N, NPAD - N)])

    pltpu.sync_copy(src_hbm.at[pl.ds(wid * CW, CW)], sidx)
    pltpu.sync_copy(dst_hbm.at[pl.ds(wid * CW, CW)], didx)
    plsc.subcore_barrier()

    NB = 2
    B = 4
    yvs = (yv0, yv1, yv2, yv3)
    gsems = (gsem0, gsem1, gsem2, gsem3)
    ssems = (ssem0, ssem1, ssem2, ssem3)
    gd = [None] * B
    sd = [None] * B
    for g in range(NB):
        gd[g] = pltpu.async_copy(y_hbm.at[sidx.at[g]], yvs[g], gsems[g])
    for g in range(CW):
        b = g % B
        gd[b].wait()
        sd[b] = pltpu.async_copy(yvs[b], acc.at[didx.at[g]], ssems[b],
                                 add=True)
        n = g + NB
        if n < CW:
            bn = n % B
            if sd[bn] is not None:
                sd[bn].wait()
            gd[bn] = pltpu.async_copy(y_hbm.at[sidx.at[n]], yvs[bn],
                                      gsems[bn])
    for b in range(B):
        if sd[b] is not None:
            sd[b].wait()
            sd[b] = None

    plsc.subcore_barrier()

    @pl.when(s < NS - 1)
    def _():
        pltpu.sync_copy(acc.at[pl.ds(s * RPS, RPS)], zbuf.at[pl.ds(0, RPS)])
        pltpu.sync_copy(zbuf.at[pl.ds(0, RPS)],
                        out_hbm.at[pl.ds(c * N + s * RPS, RPS)])

    @pl.when(s == NS - 1)
    def _():
        pltpu.sync_copy(acc.at[pl.ds(15 * RPS, RPS_LAST)], zbuf)
        pltpu.sync_copy(zbuf,
                        out_hbm.at[pl.ds(c * N + 15 * RPS, RPS_LAST)])


_sc2 = pl.kernel(
    _sc_segsum_scalar,
    out_type=jax.ShapeDtypeStruct((NC * N,), jnp.float32),
    mesh=_mesh,
    scratch_types=[
        pltpu.VMEM_SHARED((NPAD,), jnp.float32),
        pltpu.VMEM((CW, CHUNK), jnp.int32),
        pltpu.VMEM((CW, CHUNK), jnp.int32),
        pltpu.VMEM((CHUNK,), jnp.float32),
        pltpu.VMEM((CHUNK,), jnp.float32),
        pltpu.VMEM((CHUNK,), jnp.float32),
        pltpu.VMEM((CHUNK,), jnp.float32),
        pltpu.VMEM((RPS_LAST,), jnp.float32),
    ] + [pltpu.SemaphoreType.DMA] * 8,
)

_BM = 1000


def _tc_dense_body(p0, p1, x, w1rel, w1root, b1, w2rel_t, w2root_t,
                   y_out, r2_out):
    agg = p0[...] + p1[...]
    h = jnp.dot(agg, w1rel[...], preferred_element_type=jnp.float32)
    h = h + jnp.dot(x[...], w1root[...], preferred_element_type=jnp.float32)
    h = jnp.maximum(h + b1[...], 0.0)
    y_out[...] = jnp.sum(h * w2rel_t[...], axis=1, keepdims=True)
    r2_out[...] = jnp.sum(h * w2root_t[...], axis=1, keepdims=True)


def _tc_out_body(s0, s1, r2, b2, o):
    o[...] = jax.nn.sigmoid(s0[...] + s1[...] + r2[...] + b2[...])


def kernel(x, edge_index, W1_rel, W1_root, b1, W2_rel, W2_root, b2):
    pad = EP - E
    src = jnp.concatenate([edge_index[0], jnp.zeros((pad,), jnp.int32)])
    dst = jnp.concatenate([edge_index[1], jnp.full((pad,), N, jnp.int32)])
    src2d = src.reshape(NCHT, CHUNK)
    dst2d = dst.reshape(NCHT, CHUNK)

    parts = _sc1(x, src, dst)

    full = pl.BlockSpec((D, D), lambda i: (0, 0))
    row1 = pl.BlockSpec((1, D), lambda i: (0, 0))
    blk = pl.BlockSpec((_BM, D), lambda i: (i, 0))
    col = pl.BlockSpec((_BM, 1), lambda i: (i, 0))
    y, r2 = pl.pallas_call(
        _tc_dense_body,
        grid=(N // _BM,),
        in_specs=[blk, blk, blk, full, full, row1, row1, row1],
        out_specs=[col, col],
        out_shape=[
            jax.ShapeDtypeStruct((N, 1), jnp.float32),
            jax.ShapeDtypeStruct((N, 1), jnp.float32),
        ],
    )(parts[0], parts[1], x, W1_rel, W1_root, b1.reshape(1, D),
      W2_rel.reshape(1, D), W2_root.reshape(1, D))

    sparts = _sc2(y.reshape(N), src2d, dst2d)

    one = pl.BlockSpec((1, 1), lambda i: (0, 0))
    out = pl.pallas_call(
        _tc_out_body,
        grid=(N // _BM,),
        in_specs=[col, col, col, one],
        out_specs=col,
        out_shape=jax.ShapeDtypeStruct((N, 1), jnp.float32),
    )(sparts[:N].reshape(N, 1), sparts[N:].reshape(N, 1), r2,
      b2.reshape(1, 1))
    return out


# speedup vs baseline: 1.1464x; 1.1464x over previous
import jax
import jax.numpy as jnp
from jax import lax
from jax.experimental import pallas as pl
from jax.experimental.pallas import tpu as pltpu
from jax.experimental.pallas import tpu_sc as plsc

N = 10000
E = 320000
D = 128
NC = 2
NS = 16
NW = NC * NS
CHUNK = 128
NCHT = 2560
EP = NCHT * CHUNK
CW = NCHT // NW
FASTC = 1
CW_FAST = 122
CW_SLOW = 38
NPAD = N + 16
RPS = 624
RPS_LAST = N - 15 * RPS

_mesh = plsc.VectorSubcoreMesh(
    core_axis_name="c", subcore_axis_name="s", num_cores=NC, num_subcores=NS
)


def _sc_segsum_wide(x_hbm, src_hbm, dst_hbm, out_hbm, acc,
                    sidx0, sidx1, sidx2, didx0, didx1, didx2,
                    rows0, rows1, rows2,
                    isem0, isem1, isem2, jsem0, jsem1, jsem2,
                    gsem0, gsem1, gsem2, ssem0, ssem1, ssem2):
    c = lax.axis_index("c")
    s = lax.axis_index("s")
    is_fast = c == FASTC
    cw = jnp.where(is_fast, CW_FAST, CW_SLOW)
    e0 = jnp.where(is_fast, s * CW_FAST,
                   NS * CW_FAST + s * CW_SLOW) * CHUNK

    sidx = (sidx0, sidx1, sidx2)
    didx = (didx0, didx1, didx2)
    rows = (rows0, rows1, rows2)
    isem = (isem0, isem1, isem2)
    jsem = (jsem0, jsem1, jsem2)
    gsem = (gsem0, gsem1, gsem2)
    ssem = (ssem0, ssem1, ssem2)

    zero16 = jnp.zeros((16,), jnp.float32)

    def _zrow(i, carry):
        for k in range(D // 16):
            rows0[i, pl.ds(k * 16, 16)] = zero16
        return carry

    lax.fori_loop(0, CHUNK, _zrow, 0)
    r0 = s * RPS

    @pl.when(s < NS - 1)
    def _():
        off = 0
        for m in (128, 128, 128, 128, RPS - 4 * 128):
            pltpu.sync_copy(rows0.at[pl.ds(0, m)],
                            acc.at[pl.ds(r0 + off, m)])
            off += m

    @pl.when(s == NS - 1)
    def _():
        for k in range(5):
            pltpu.sync_copy(rows0, acc.at[pl.ds(15 * RPS + k * CHUNK, CHUNK)])
        pltpu.sync_copy(rows0.at[pl.ds(0, NPAD - N)],
                        acc.at[pl.ds(N, NPAD - N)])

    plsc.subcore_barrier()

    def idx_issue(g, b):
        pltpu.async_copy(src_hbm.at[pl.ds(e0 + g * CHUNK, CHUNK)],
                         sidx[b], isem[b])
        pltpu.async_copy(dst_hbm.at[pl.ds(e0 + g * CHUNK, CHUNK)],
                         didx[b], jsem[b])

    def idx_wait(b):
        pltpu.make_async_copy(src_hbm.at[pl.ds(e0, CHUNK)], sidx[b],
                              isem[b]).wait()
        pltpu.make_async_copy(dst_hbm.at[pl.ds(e0, CHUNK)], didx[b],
                              jsem[b]).wait()

    def gather_issue(b):
        pltpu.async_copy(x_hbm.at[sidx[b]], rows[b], gsem[b])

    def gather_wait(b):
        pltpu.make_async_copy(x_hbm.at[sidx[b]], rows[b], gsem[b]).wait()

    def scatter_issue(b):
        pltpu.async_copy(rows[b], acc.at[didx[b]], ssem[b], add=True)

    def scatter_wait(b):
        pltpu.make_async_copy(rows[b], acc.at[didx[b]], ssem[b]).wait()

    idx_issue(0, 0)
    idx_issue(1, 1)
    idx_wait(0)
    gather_issue(0)
    gather_wait(0)
    scatter_issue(0)
    idx_issue(2, 2)
    idx_wait(1)
    gather_issue(1)

    def _steady(outer, carry):
        g1 = 1 + outer * 3
        for u in range(3):
            g = g1 + u
            b0 = (1 + u) % 3
            b2 = (u + 3) % 3
            b1 = (2 + u) % 3
            gather_wait(b0)
            scatter_issue(b0)
            scatter_wait(b2)
            idx_issue(g + 2, b2)
            idx_wait(b1)
            gather_issue(b1)
        return carry

    lax.fori_loop(0, (cw - 5) // 3, _steady, 0)

    for k in range(4):
        b0 = (1 + k) % 3
        b2 = (k + 3) % 3
        b1 = (2 + k) % 3
        g = cw - 4 + k
        gather_wait(b0)
        scatter_issue(b0)
        scatter_wait(b2)
        if k < 2:
            idx_issue(g + 2, b2)
        if k < 3:
            idx_wait(b1)
            gather_issue(b1)
    scatter_wait(1)

    plsc.subcore_barrier()

    @pl.when(s < NS - 1)
    def _():
        pltpu.sync_copy(acc.at[pl.ds(r0, RPS)], out_hbm.at[c, pl.ds(r0, RPS)])

    @pl.when(s == NS - 1)
    def _():
        pltpu.sync_copy(acc.at[pl.ds(15 * RPS, RPS_LAST)],
                        out_hbm.at[c, pl.ds(15 * RPS, RPS_LAST)])


_sc1 = pl.kernel(
    _sc_segsum_wide,
    out_type=jax.ShapeDtypeStruct((NC, N, D), jnp.float32),
    mesh=_mesh,
    scratch_types=[
        pltpu.VMEM_SHARED((NPAD, D), jnp.float32),
        pltpu.VMEM((CHUNK,), jnp.int32),
        pltpu.VMEM((CHUNK,), jnp.int32),
        pltpu.VMEM((CHUNK,), jnp.int32),
        pltpu.VMEM((CHUNK,), jnp.int32),
        pltpu.VMEM((CHUNK,), jnp.int32),
        pltpu.VMEM((CHUNK,), jnp.int32),
        pltpu.VMEM((CHUNK, D), jnp.float32),
        pltpu.VMEM((CHUNK, D), jnp.float32),
        pltpu.VMEM((CHUNK, D), jnp.float32),
    ] + [pltpu.SemaphoreType.DMA] * 12,
)


def _sc_segsum_scalar(y_hbm, src_hbm, dst_hbm, out_hbm, acc, sidx, didx,
                      yv0, yv1, yv2, yv3, zbuf,
                      gsem0, gsem1, gsem2, gsem3,
                      ssem0, ssem1, ssem2, ssem3):
    c = lax.axis_index("c")
    s = lax.axis_index("s")
    wid = s * NC + c

    zero16 = jnp.zeros((16,), jnp.float32)

    def _z(i, carry):
        zbuf[pl.ds(i * 16, 16)] = zero16
        return carry

    lax.fori_loop(0, RPS_LAST // 16, _z, 0)

    @pl.when(s < NS - 1)
    def _():
        pltpu.sync_copy(zbuf.at[pl.ds(0, RPS)], acc.at[pl.ds(s * RPS, RPS)])

    @pl.when(s == NS - 1)
    def _():
        pltpu.sync_copy(zbuf, acc.at[pl.ds(15 * RPS, RPS_LAST)])
        pltpu.sync_copy(zbuf.at[pl.ds(0, NPAD - N)], acc.at[pl.ds(N, NPAD - N)])

    pltpu.sync_copy(src_hbm.at[pl.ds(wid * CW, CW)], sidx)
    pltpu.sync_copy(dst_hbm.at[pl.ds(wid * CW, CW)], didx)
    plsc.subcore_barrier()

    NB = 2
    B = 4
    yvs = (yv0, yv1, yv2, yv3)
    gsems = (gsem0, gsem1, gsem2, gsem3)
    ssems = (ssem0, ssem1, ssem2, ssem3)
    gd = [None] * B
    sd = [None] * B
    for g in range(NB):
        gd[g] = pltpu.async_copy(y_hbm.at[sidx.at[g]], yvs[g], gsems[g])
    for g in range(CW):
        b = g % B
        gd[b].wait()
        sd[b] = pltpu.async_copy(yvs[b], acc.at[didx.at[g]], ssems[b],
                                 add=True)
        n = g + NB
        if n < CW:
            bn = n % B
            if sd[bn] is not None:
                sd[bn].wait()
            gd[bn] = pltpu.async_copy(y_hbm.at[sidx.at[n]], yvs[bn],
                                      gsems[bn])
    for b in range(B):
        if sd[b] is not None:
            sd[b].wait()
            sd[b] = None

    plsc.subcore_barrier()

    @pl.when(s < NS - 1)
    def _():
        pltpu.sync_copy(acc.at[pl.ds(s * RPS, RPS)], zbuf.at[pl.ds(0, RPS)])
        pltpu.sync_copy(zbuf.at[pl.ds(0, RPS)],
                        out_hbm.at[pl.ds(c * N + s * RPS, RPS)])

    @pl.when(s == NS - 1)
    def _():
        pltpu.sync_copy(acc.at[pl.ds(15 * RPS, RPS_LAST)], zbuf)
        pltpu.sync_copy(zbuf,
                        out_hbm.at[pl.ds(c * N + 15 * RPS, RPS_LAST)])


_sc2 = pl.kernel(
    _sc_segsum_scalar,
    out_type=jax.ShapeDtypeStruct((NC * N,), jnp.float32),
    mesh=_mesh,
    scratch_types=[
        pltpu.VMEM_SHARED((NPAD,), jnp.float32),
        pltpu.VMEM((CW, CHUNK), jnp.int32),
        pltpu.VMEM((CW, CHUNK), jnp.int32),
        pltpu.VMEM((CHUNK,), jnp.float32),
        pltpu.VMEM((CHUNK,), jnp.float32),
        pltpu.VMEM((CHUNK,), jnp.float32),
        pltpu.VMEM((CHUNK,), jnp.float32),
        pltpu.VMEM((RPS_LAST,), jnp.float32),
    ] + [pltpu.SemaphoreType.DMA] * 8,
)

_BM = 1000


def _tc_dense_body(p0, p1, x, w1rel, w1root, b1, w2rel_t, w2root_t,
                   y_out, r2_out):
    agg = p0[...] + p1[...]
    h = jnp.dot(agg, w1rel[...], preferred_element_type=jnp.float32)
    h = h + jnp.dot(x[...], w1root[...], preferred_element_type=jnp.float32)
    h = jnp.maximum(h + b1[...], 0.0)
    y_out[...] = jnp.sum(h * w2rel_t[...], axis=1, keepdims=True)
    r2_out[...] = jnp.sum(h * w2root_t[...], axis=1, keepdims=True)


def _tc_out_body(s0, s1, r2, b2, o):
    o[...] = jax.nn.sigmoid(s0[...] + s1[...] + r2[...] + b2[...])


def kernel(x, edge_index, W1_rel, W1_root, b1, W2_rel, W2_root, b2):
    pad = EP - E
    src = jnp.concatenate([edge_index[0], jnp.zeros((pad,), jnp.int32)])
    dst = jnp.concatenate([edge_index[1], jnp.full((pad,), N, jnp.int32)])
    src2d = src.reshape(NCHT, CHUNK)
    dst2d = dst.reshape(NCHT, CHUNK)

    parts = _sc1(x, src, dst)

    full = pl.BlockSpec((D, D), lambda i: (0, 0))
    row1 = pl.BlockSpec((1, D), lambda i: (0, 0))
    blk = pl.BlockSpec((_BM, D), lambda i: (i, 0))
    col = pl.BlockSpec((_BM, 1), lambda i: (i, 0))
    y, r2 = pl.pallas_call(
        _tc_dense_body,
        grid=(N // _BM,),
        in_specs=[blk, blk, blk, full, full, row1, row1, row1],
        out_specs=[col, col],
        out_shape=[
            jax.ShapeDtypeStruct((N, 1), jnp.float32),
            jax.ShapeDtypeStruct((N, 1), jnp.float32),
        ],
    )(parts[0], parts[1], x, W1_rel, W1_root, b1.reshape(1, D),
      W2_rel.reshape(1, D), W2_root.reshape(1, D))

    sparts = _sc2(y.reshape(N), src2d, dst2d)

    one = pl.BlockSpec((1, 1), lambda i: (0, 0))
    out = pl.pallas_call(
        _tc_out_body,
        grid=(N // _BM,),
        in_specs=[col, col, col, one],
        out_specs=col,
        out_shape=jax.ShapeDtypeStruct((N, 1), jnp.float32),
    )(sparts[:N].reshape(N, 1), sparts[N:].reshape(N, 1), r2,
      b2.reshape(1, 1))
    return out
